# baseline (speedup 1.0000x reference)
import dataclasses

import jax
import jax.numpy as jnp
from jax.experimental import pallas as pl
from jax.experimental.pallas import tpu as pltpu
from jax.experimental.pallas import tpu_sc as plsc
from jax.experimental.layout import Format, Layout, with_layout_constraint

_SC_COMPILER_PARAMS = pltpu.CompilerParams()
if "needs_layout_passes" in pltpu.CompilerParams.__dataclass_fields__:
    _SC_COMPILER_PARAMS = dataclasses.replace(
        _SC_COMPILER_PARAMS, needs_layout_passes=False)

B = 16384
F = 26
V = 100000
D = 16
NC = 13
H1 = 128
H2 = 64
OUT = 1
EPS = 1e-5

GW = 128
BB = 1024


def _sc_gather(tables_flat, idx_flat):
    mesh = plsc.VectorSubcoreMesh(core_axis_name="core", subcore_axis_name="subcore")
    n = B * F
    NW = 32
    SUB = 128
    K = 4
    W = K * SUB
    NIT = n // NW // W
    WD = 8 * D

    @pl.kernel(
        out_type=jax.ShapeDtypeStruct((n, WD), jnp.float32),
        mesh=mesh,
        scratch_types=[
            pltpu.VMEM((W,), jnp.int32),
            pltpu.SemaphoreType.DMA,
            pltpu.SemaphoreType.DMA,
        ] + [pltpu.VMEM((SUB, WD), jnp.float32) for _ in range(K)],
        compiler_params=_SC_COMPILER_PARAMS,
    )
    def gather_kernel(tab_hbm, idx_hbm, out_hbm, idx_v, sem, osem, *rows):
        w = jax.lax.axis_index("subcore") * 2 + jax.lax.axis_index("core")
        base_w = w * NIT

        @pl.loop(0, NIT)
        def _(g):
            slot = (base_w + g) * W
            pltpu.sync_copy(idx_hbm.at[pl.ds(slot, W)], idx_v)
            gathers = [
                pltpu.async_copy(
                    tab_hbm.at[idx_v.at[pl.ds(SUB * t, SUB)]], rows[t], sem)
                for t in range(K)
            ]
            outs = []
            for t in range(K):
                gathers[t].wait()
                outs.append(pltpu.async_copy(
                    rows[t], out_hbm.at[pl.ds(slot + SUB * t, SUB)], osem))
            for o in outs:
                o.wait()

    return gather_kernel(tables_flat, idx_flat)


def _mlp_body(xc_ref, xn_ref, w1e_ref, w1c_ref, b1_ref, w2_ref, b2_ref,
              w3_ref, b3_ref, out_ref):
    h = (
        jnp.dot(xc_ref[...], w1e_ref[...], preferred_element_type=jnp.float32)
        + jnp.dot(xn_ref[...], w1c_ref[...], preferred_element_type=jnp.float32)
        + b1_ref[...]
    )
    h = jnp.maximum(h, 0.0)
    h = jnp.dot(h, w2_ref[...], preferred_element_type=jnp.float32) + b2_ref[...]
    h = jnp.maximum(h, 0.0)
    out_ref[...] = (
        jnp.dot(h, w3_ref[...], preferred_element_type=jnp.float32) + b3_ref[...]
    )


def _mlp(xc, x_cont, w1e, w1c, b1f, w2f, b2f, w3f, b3f):
    n_emb = F * D
    grid = (B // BB,)
    full = lambda i: (0, 0)
    return pl.pallas_call(
        _mlp_body,
        grid=grid,
        in_specs=[
            pl.BlockSpec((BB, n_emb), lambda i: (i, 0)),
            pl.BlockSpec((BB, NC), lambda i: (i, 0)),
            pl.BlockSpec((n_emb, H1), full),
            pl.BlockSpec((NC, H1), full),
            pl.BlockSpec((1, H1), full),
            pl.BlockSpec((H1, H2), full),
            pl.BlockSpec((1, H2), full),
            pl.BlockSpec((H2, OUT), full),
            pl.BlockSpec((1, OUT), full),
        ],
        out_specs=pl.BlockSpec((BB, OUT), lambda i: (i, 0)),
        out_shape=jax.ShapeDtypeStruct((B, OUT), jnp.float32),
    )(xc, x_cont, w1e, w1c, b1f, w2f, b2f, w3f, b3f)


def kernel(x_cat, x_cont, tables, gamma_c, beta_c,
           W1, b1, g1, bb1, W2, b2, g2, bb2, W3, b3):
    inv = 1.0 / jnp.sqrt(jnp.float32(1.0 + EPS))
    n_emb = F * D

    w1e = W1[:n_emb]
    w1c = (gamma_c * inv)[:, None] * W1[n_emb:]
    b1f = (b1 + beta_c @ W1[n_emb:])[None, :]
    w2f = (g1 * inv)[:, None] * W2
    b2f = (b2 + bb1 @ W2)[None, :]
    w3f = (g2 * inv)[:, None] * W3
    b3f = (b3 + bb2 @ W3)[None, :]

    offs = (jnp.arange(F, dtype=jnp.int32) * V)[None, :]
    idx_flat = (x_cat.astype(jnp.int32) + offs).reshape(-1)

    tables_g = tables.reshape(F * V // 8, 8 * D)
    raw = _sc_gather(tables_g, idx_flat // 8)
    sel = (idx_flat % 8).astype(jnp.int32)
    xc = jnp.take_along_axis(
        raw.reshape(B * F, 8, D), sel[:, None, None], axis=1)[:, 0]
    xc = xc.reshape(B, F * D)
    return _mlp(xc, x_cont, w1e, w1c, b1f, w2f, b2f, w3f, b3f)


# speedup vs baseline: 1.3987x; 1.3987x over previous
import dataclasses

import jax
import jax.numpy as jnp
from jax.experimental import pallas as pl
from jax.experimental.pallas import tpu as pltpu
from jax.experimental.pallas import tpu_sc as plsc
from jax.experimental.layout import Format, Layout, with_layout_constraint

_SC_COMPILER_PARAMS = pltpu.CompilerParams()
if "needs_layout_passes" in pltpu.CompilerParams.__dataclass_fields__:
    _SC_COMPILER_PARAMS = dataclasses.replace(
        _SC_COMPILER_PARAMS, needs_layout_passes=False)

B = 16384
F = 26
V = 100000
D = 16
NC = 13
H1 = 128
H2 = 64
OUT = 1
EPS = 1e-5

GW = 128
BB = 1024


def _sc_gather(tables_flat, idx_flat):
    mesh = plsc.VectorSubcoreMesh(core_axis_name="core", subcore_axis_name="subcore")
    n = B * F
    NW = 32
    SUB = 128
    K = 4
    W = K * SUB
    NIT = 8 * n // NW // W

    @pl.kernel(
        out_type=jax.ShapeDtypeStruct((8 * n, D), jnp.float32),
        mesh=mesh,
        scratch_types=[
            pltpu.VMEM((W,), jnp.int32),
            pltpu.SemaphoreType.DMA,
            pltpu.SemaphoreType.DMA,
        ] + [pltpu.VMEM((SUB, D), jnp.float32) for _ in range(K)],
        compiler_params=_SC_COMPILER_PARAMS,
    )
    def gather_kernel(tab_hbm, idx_hbm, out_hbm, idx_v, sem, osem, *rows):
        w = jax.lax.axis_index("subcore") * 2 + jax.lax.axis_index("core")
        base_w = w * NIT

        @pl.loop(0, NIT)
        def _(g):
            slot = (base_w + g) * W
            pltpu.sync_copy(idx_hbm.at[pl.ds(slot, W)], idx_v)
            gathers = [
                pltpu.async_copy(
                    tab_hbm.at[idx_v.at[pl.ds(SUB * t, SUB)]], rows[t], sem)
                for t in range(K)
            ]
            outs = []
            for t in range(K):
                gathers[t].wait()
                outs.append(pltpu.async_copy(
                    rows[t], out_hbm.at[pl.ds(slot + SUB * t, SUB)], osem))
            for o in outs:
                o.wait()

    raw = gather_kernel(tables_flat, jnp.repeat(idx_flat, 8))
    return raw.reshape(8 * n // SUB, SUB, D)[:, :SUB // 8].reshape(n, D)


def _mlp_body(xc_ref, xn_ref, w1e_ref, w1c_ref, b1_ref, w2_ref, b2_ref,
              w3_ref, b3_ref, out_ref):
    h = (
        jnp.dot(xc_ref[...], w1e_ref[...], preferred_element_type=jnp.float32)
        + jnp.dot(xn_ref[...], w1c_ref[...], preferred_element_type=jnp.float32)
        + b1_ref[...]
    )
    h = jnp.maximum(h, 0.0)
    h = jnp.dot(h, w2_ref[...], preferred_element_type=jnp.float32) + b2_ref[...]
    h = jnp.maximum(h, 0.0)
    out_ref[...] = (
        jnp.dot(h, w3_ref[...], preferred_element_type=jnp.float32) + b3_ref[...]
    )


def _mlp(xc, x_cont, w1e, w1c, b1f, w2f, b2f, w3f, b3f):
    n_emb = F * D
    grid = (B // BB,)
    full = lambda i: (0, 0)
    return pl.pallas_call(
        _mlp_body,
        grid=grid,
        in_specs=[
            pl.BlockSpec((BB, n_emb), lambda i: (i, 0)),
            pl.BlockSpec((BB, NC), lambda i: (i, 0)),
            pl.BlockSpec((n_emb, H1), full),
            pl.BlockSpec((NC, H1), full),
            pl.BlockSpec((1, H1), full),
            pl.BlockSpec((H1, H2), full),
            pl.BlockSpec((1, H2), full),
            pl.BlockSpec((H2, OUT), full),
            pl.BlockSpec((1, OUT), full),
        ],
        out_specs=pl.BlockSpec((BB, OUT), lambda i: (i, 0)),
        out_shape=jax.ShapeDtypeStruct((B, OUT), jnp.float32),
    )(xc, x_cont, w1e, w1c, b1f, w2f, b2f, w3f, b3f)


def kernel(x_cat, x_cont, tables, gamma_c, beta_c,
           W1, b1, g1, bb1, W2, b2, g2, bb2, W3, b3):
    inv = 1.0 / jnp.sqrt(jnp.float32(1.0 + EPS))
    n_emb = F * D

    w1e = W1[:n_emb]
    w1c = (gamma_c * inv)[:, None] * W1[n_emb:]
    b1f = (b1 + beta_c @ W1[n_emb:])[None, :]
    w2f = (g1 * inv)[:, None] * W2
    b2f = (b2 + bb1 @ W2)[None, :]
    w3f = (g2 * inv)[:, None] * W3
    b3f = (b3 + bb2 @ W3)[None, :]

    offs = (jnp.arange(F, dtype=jnp.int32) * V)[None, :]
    idx_flat = (x_cat.astype(jnp.int32) + offs).reshape(-1)

    tables_flat = with_layout_constraint(
        tables.reshape(F * V, D),
        Layout(major_to_minor=(0, 1), tiling=((16,),)),
    )
    xc = _sc_gather(tables_flat, idx_flat * 8).reshape(B, F * D)
    return _mlp(xc, x_cont, w1e, w1c, b1f, w2f, b2f, w3f, b3f)
